# baseline (speedup 1.0000x reference)
import functools

import jax
import jax.numpy as jnp
from jax import lax
from jax.experimental import pallas as pl
from jax.experimental.pallas import tpu as pltpu
from jax.experimental.pallas import tpu_sc as plsc

_LANES = 128


def _sc_dims():
    try:
        info = plsc.get_sparse_core_info()
        return info.num_cores, info.num_subcores
    except Exception:
        return 2, 16


@functools.lru_cache(maxsize=None)
def _make_sc_gather(B, QP, NC, NS):
    NW = NC * NS
    CH = B // NW // _LANES
    QR = QP // _LANES
    mesh = plsc.VectorSubcoreMesh(core_axis_name="c", subcore_axis_name="s")

    @functools.partial(
        pl.kernel,
        out_type=(
            jax.ShapeDtypeStruct((B,), jnp.float32),
            jax.ShapeDtypeStruct((QR, _LANES), jnp.float32),
        ),
        mesh=mesh,
        scratch_types=(
            pltpu.VMEM((CH, _LANES), jnp.int32),
            pltpu.VMEM((CH, _LANES), jnp.float32),
            pltpu.VMEM((_LANES,), jnp.int32),
            pltpu.VMEM((_LANES,), jnp.float32),
            pltpu.SemaphoreType.DMA,
            pltpu.SemaphoreType.DMA,
        ),
    )
    def gather(stud_hbm, ques_hbm, abil_hbm, diff_hbm, sv_hbm, dv_hbm,
               sidx, srow, qidx, qrow, sem_s, sem_q):
        wid = lax.axis_index("s") * NC + lax.axis_index("c")
        pltpu.sync_copy(stud_hbm.at[wid], sidx)
        copies = [pltpu.async_copy(abil_hbm.at[sidx.at[j]], srow.at[j], sem_s)
                  for j in range(CH)]

        @pl.when(wid < QR)
        def _():
            pltpu.sync_copy(ques_hbm.at[wid], qidx)
            pltpu.async_copy(diff_hbm.at[qidx], qrow, sem_q).wait()
            pltpu.sync_copy(qrow, dv_hbm.at[wid])

        for c in copies:
            c.wait()
        for j in range(CH):
            pltpu.sync_copy(
                srow.at[j],
                sv_hbm.at[pl.ds(wid * CH * _LANES + j * _LANES, _LANES)])

    return gather


@functools.lru_cache(maxsize=None)
def _make_tc_dense(B, Q, QP, BS):
    QR = QP // _LANES

    def body(sv_ref, dv_ref, o_ref):
        sv = sv_ref[...]
        d = dv_ref[...]
        qi = jax.lax.broadcasted_iota(jnp.int32, (Q, QR), 0) // _LANES
        ji = jax.lax.broadcasted_iota(jnp.int32, (Q, QR), 1)
        p1 = (qi == ji).astype(jnp.float32)
        y = jnp.dot(p1, d, preferred_element_type=jnp.float32)
        ql = jax.lax.broadcasted_iota(jnp.int32, (Q, _LANES), 0) % _LANES
        ll = jax.lax.broadcasted_iota(jnp.int32, (Q, _LANES), 1)
        col = jnp.sum(jnp.where(ql == ll, y, 0.0), axis=1, keepdims=True)
        o_ref[...] = jax.nn.sigmoid(sv[None, :] - col)

    return pl.pallas_call(
        body,
        grid=(B // BS,),
        in_specs=[
            pl.BlockSpec((BS,), lambda i: (i,)),
            pl.BlockSpec((QR, _LANES), lambda i: (0, 0)),
        ],
        out_specs=pl.BlockSpec((Q, BS), lambda i: (0, i)),
        out_shape=jax.ShapeDtypeStruct((Q, B), jnp.float32),
    )


def kernel(students, questions, student_abilities, question_difficulties):
    B = students.shape[0]
    Q = questions.shape[0]
    NC, NS = _sc_dims()
    NW = NC * NS
    CH = B // NW // _LANES

    QP = -(-Q // _LANES) * _LANES

    stud = students.astype(jnp.int32).reshape(NW, CH, _LANES)
    ques = jnp.pad(questions.astype(jnp.int32), (0, QP - Q)).reshape(-1, _LANES)
    abil = student_abilities.reshape(-1)
    diff = question_difficulties.reshape(-1)

    sv, dv = _make_sc_gather(B, QP, NC, NS)(stud, ques, abil, diff)
    out_t = _make_tc_dense(B, Q, QP, 8192)(sv, dv)
    return jnp.transpose(out_t)


# speedup vs baseline: 1.0009x; 1.0009x over previous
import functools

import jax
import jax.numpy as jnp
from jax import lax
from jax.experimental import pallas as pl
from jax.experimental.pallas import tpu as pltpu
from jax.experimental.pallas import tpu_sc as plsc

_LANES = 128


def _sc_dims():
    try:
        info = plsc.get_sparse_core_info()
        return info.num_cores, info.num_subcores
    except Exception:
        return 2, 16


@functools.lru_cache(maxsize=None)
def _make_sc_gather(B, QP, NC, NS):
    NW = NC * NS
    CH = B // NW // _LANES
    QR = QP // _LANES
    mesh = plsc.VectorSubcoreMesh(core_axis_name="c", subcore_axis_name="s")

    @functools.partial(
        pl.kernel,
        out_type=(
            jax.ShapeDtypeStruct((B,), jnp.float32),
            jax.ShapeDtypeStruct((QR, _LANES), jnp.float32),
        ),
        mesh=mesh,
        scratch_types=(
            pltpu.VMEM((CH, _LANES), jnp.int32),
            pltpu.VMEM((CH, _LANES), jnp.float32),
            pltpu.VMEM((_LANES,), jnp.int32),
            pltpu.VMEM((_LANES,), jnp.float32),
            pltpu.SemaphoreType.DMA,
            pltpu.SemaphoreType.DMA,
        ),
    )
    def gather(stud_hbm, ques_hbm, abil_hbm, diff_hbm, sv_hbm, dv_hbm,
               sidx, srow, qidx, qrow, sem_s, sem_q):
        wid = lax.axis_index("s") * NC + lax.axis_index("c")
        pltpu.sync_copy(stud_hbm.at[wid], sidx)
        copies = [pltpu.async_copy(abil_hbm.at[sidx.at[j]], srow.at[j], sem_s)
                  for j in range(CH)]

        @pl.when(wid < QR)
        def _():
            pltpu.sync_copy(ques_hbm.at[wid], qidx)
            pltpu.async_copy(diff_hbm.at[qidx], qrow, sem_q).wait()
            pltpu.sync_copy(qrow, dv_hbm.at[wid])

        for c in copies:
            c.wait()
        for j in range(CH):
            pltpu.sync_copy(
                srow.at[j],
                sv_hbm.at[pl.ds(wid * CH * _LANES + j * _LANES, _LANES)])

    return gather


@functools.lru_cache(maxsize=None)
def _make_tc_dense(B, Q, QP, BS):
    QR = QP // _LANES

    def body(sv_ref, dv_ref, o_ref):
        sv = sv_ref[...]
        d = dv_ref[...]
        qi = jax.lax.broadcasted_iota(jnp.int32, (Q, QR), 0) // _LANES
        ji = jax.lax.broadcasted_iota(jnp.int32, (Q, QR), 1)
        p1 = (qi == ji).astype(jnp.float32)
        y = jnp.dot(p1, d, preferred_element_type=jnp.float32)
        ql = jax.lax.broadcasted_iota(jnp.int32, (Q, _LANES), 0) % _LANES
        ll = jax.lax.broadcasted_iota(jnp.int32, (Q, _LANES), 1)
        col = jnp.sum(jnp.where(ql == ll, y, 0.0), axis=1, keepdims=True)
        o_ref[...] = jax.nn.sigmoid(sv[None, :] - col)

    return pl.pallas_call(
        body,
        grid=(B // BS,),
        in_specs=[
            pl.BlockSpec((BS,), lambda i: (i,)),
            pl.BlockSpec((QR, _LANES), lambda i: (0, 0)),
        ],
        out_specs=pl.BlockSpec((Q, BS), lambda i: (0, i)),
        out_shape=jax.ShapeDtypeStruct((Q, B), jnp.float32),
    )


def kernel(students, questions, student_abilities, question_difficulties):
    B = students.shape[0]
    Q = questions.shape[0]
    NC, NS = _sc_dims()
    NW = NC * NS
    CH = B // NW // _LANES

    QP = -(-Q // _LANES) * _LANES

    stud = students.astype(jnp.int32).reshape(NW, CH, _LANES)
    ques = jnp.pad(questions.astype(jnp.int32), (0, QP - Q)).reshape(-1, _LANES)
    abil = student_abilities.reshape(-1)
    diff = question_difficulties.reshape(-1)

    sv, dv = _make_sc_gather(B, QP, NC, NS)(stud, ques, abil, diff)
    out_t = _make_tc_dense(B, Q, QP, 4096)(sv, dv)
    return jnp.transpose(out_t)
